# baseline (speedup 1.0000x reference)
import jax
import jax.numpy as jnp
from jax import lax
from jax.experimental import pallas as pl
from jax.experimental.pallas import tpu as pltpu
from jax.experimental.pallas import tpu_sc as plsc

NUM_FIELDS = 26
VOCAB = 100000
EMB = 32
BATCH = 16384
D_IN = NUM_FIELDS * EMB
H1, H2 = 256, 128
EPS = 1e-5

IDX_CHUNK = 4096
N_CHUNKS = BATCH // IDX_CHUNK


def _sc_gather_t(t2T, t1T, xT_idx):
    mesh = plsc.VectorSubcoreMesh(core_axis_name="c", subcore_axis_name="s")

    @pl.kernel(
        out_type=(
            jax.ShapeDtypeStruct((D_IN, BATCH), jnp.float32),
            jax.ShapeDtypeStruct((NUM_FIELDS, BATCH), jnp.float32),
        ),
        mesh=mesh,
        scratch_types=[
            pltpu.VMEM((VOCAB,), jnp.float32),
            pltpu.VMEM((IDX_CHUNK,), jnp.int32),
            pltpu.VMEM((IDX_CHUNK,), jnp.int32),
            pltpu.VMEM((IDX_CHUNK,), jnp.float32),
            pltpu.VMEM((IDX_CHUNK,), jnp.float32),
            pltpu.SemaphoreType.DMA,
            pltpu.SemaphoreType.DMA,
            pltpu.SemaphoreType.DMA,
            pltpu.SemaphoreType.DMA,
            pltpu.SemaphoreType.DMA,
        ],
        compiler_params=pltpu.CompilerParams(use_tc_tiling_on_sc=True,
                                             needs_layout_passes=False),
    )
    def k(t2T_hbm, t1T_hbm, idx_hbm, o2T_hbm, o1T_hbm, row_v,
          idx_v0, idx_v1, out_v0, out_v1, sem_row, sem_i0, sem_i1,
          sem_o0, sem_o1):
        w = lax.axis_index("s") * 2 + lax.axis_index("c")
        idx_bufs, out_bufs = (idx_v0, idx_v1), (out_v0, out_v1)
        isems, osems = (sem_i0, sem_i1), (sem_o0, sem_o1)

        def gather_row(table_row_ref, f, out_row_ref):
            pltpu.sync_copy(table_row_ref, row_v)
            for ci in range(N_CHUNKS):
                b = ci % 2
                pltpu.sync_copy(
                    idx_hbm.at[f, pl.ds(ci * IDX_CHUNK, IDX_CHUNK)],
                    idx_bufs[b])

                @pl.loop(0, IDX_CHUNK, step=16, unroll=8)
                def _(j):
                    idx16 = idx_bufs[b][pl.ds(j, 16)]
                    out_bufs[b][pl.ds(j, 16)] = plsc.load_gather(
                        row_v, [idx16])

                pltpu.sync_copy(
                    out_bufs[b],
                    out_row_ref.at[pl.ds(ci * IDX_CHUNK, IDX_CHUNK)])

        @pl.loop(0, NUM_FIELDS)
        def _(f):
            r = f * EMB + w
            gather_row(t2T_hbm.at[r], f, o2T_hbm.at[r])

        @pl.when(w < NUM_FIELDS)
        def _():
            gather_row(t1T_hbm.at[w], w, o1T_hbm.at[w])

    return k(t2T, t1T, xT_idx)


BB = 2048


def _tc_body(xT_ref, fm1T_ref, w1_ref, b1_ref, g1_ref, be1_ref, rm1_ref,
             rv1_ref, w2_ref, b2_ref, g2_ref, be2_ref, rm2_ref, rv2_ref,
             w3_ref, b3_ref, out_ref):
    xT = xT_ref[...]

    sum_e = xT[0:EMB, :]
    sum_sq = sum_e * sum_e
    for f in range(1, NUM_FIELDS):
        v = xT[f * EMB:(f + 1) * EMB, :]
        sum_e = sum_e + v
        sum_sq = sum_sq + v * v
    fm2 = 0.5 * jnp.sum(sum_e * sum_e - sum_sq, axis=0, keepdims=True)

    fm1 = jnp.sum(fm1T_ref[...], axis=0, keepdims=True)

    dn = (((0,), (0,)), ((), ()))

    h = lax.dot_general(w1_ref[...], xT, dn,
                        preferred_element_type=jnp.float32) + b1_ref[...]
    h = (h - rm1_ref[...]) * (g1_ref[...] * lax.rsqrt(rv1_ref[...] + EPS)) + be1_ref[...]
    h = jnp.maximum(h, 0.0)
    h = lax.dot_general(w2_ref[...], h, dn,
                        preferred_element_type=jnp.float32) + b2_ref[...]
    h = (h - rm2_ref[...]) * (g2_ref[...] * lax.rsqrt(rv2_ref[...] + EPS)) + be2_ref[...]
    h = jnp.maximum(h, 0.0)
    dnn = lax.dot_general(w3_ref[...], h, dn,
                          preferred_element_type=jnp.float32) + b3_ref[...]

    out_ref[...] = jax.nn.sigmoid(fm1 + fm2 + dnn)


def _tc_head(xT, fm1T, W1, b1, g1, be1, rm1, rv1, W2, b2, g2, be2, rm2, rv2,
             W3, b3):
    grid = (BATCH // BB,)
    full = lambda shape: pl.BlockSpec(shape, lambda i: tuple(0 for _ in shape))
    return pl.pallas_call(
        _tc_body,
        grid=grid,
        in_specs=[
            pl.BlockSpec((D_IN, BB), lambda i: (0, i)),
            pl.BlockSpec((NUM_FIELDS, BB), lambda i: (0, i)),
            full((D_IN, H1)), full((H1, 1)), full((H1, 1)), full((H1, 1)),
            full((H1, 1)), full((H1, 1)),
            full((H1, H2)), full((H2, 1)), full((H2, 1)), full((H2, 1)),
            full((H2, 1)), full((H2, 1)),
            full((H2, 1)), full((1, 1)),
        ],
        out_specs=pl.BlockSpec((1, BB), lambda i: (0, i)),
        out_shape=jax.ShapeDtypeStruct((1, BATCH), jnp.float32),
    )(xT, fm1T, W1, b1, g1, be1, rm1, rv1, W2, b2, g2, be2, rm2, rv2, W3, b3)


def kernel(X_sparse, emb1, emb2, W1, b1, g1, be1, rm1, rv1, W2, b2, g2, be2,
           rm2, rv2, W3, b3):
    t2T = jnp.transpose(emb2, (0, 2, 1)).reshape(D_IN, VOCAB)
    t1T = jnp.transpose(emb1, (0, 2, 1)).reshape(NUM_FIELDS, VOCAB)
    xT_idx = jnp.transpose(X_sparse, (1, 0)).astype(jnp.int32)

    o2T, o1T = _sc_gather_t(t2T, t1T, xT_idx)

    r = lambda a: a.reshape(-1, 1)
    out_row = _tc_head(o2T, o1T, W1, r(b1), r(g1), r(be1), r(rm1), r(rv1),
                       W2, r(b2), r(g2), r(be2), r(rm2), r(rv2), r(W3), r(b3))
    return out_row.reshape(BATCH, 1)


# speedup vs baseline: 1.5481x; 1.5481x over previous
import jax
import jax.numpy as jnp
from jax import lax
from jax.experimental import pallas as pl
from jax.experimental.pallas import tpu as pltpu
from jax.experimental.pallas import tpu_sc as plsc

NUM_FIELDS = 26
VOCAB = 100000
EMB = 32
BATCH = 16384
D_IN = NUM_FIELDS * EMB
H1, H2 = 256, 128
EPS = 1e-5

IDX_CHUNK = 4096
N_CHUNKS = BATCH // IDX_CHUNK


def _sc_gather_t(t2T, t1T, xT_idx):
    mesh = plsc.VectorSubcoreMesh(core_axis_name="c", subcore_axis_name="s")

    @pl.kernel(
        out_type=(
            jax.ShapeDtypeStruct((D_IN, BATCH), jnp.float32),
            jax.ShapeDtypeStruct((NUM_FIELDS, BATCH), jnp.float32),
        ),
        mesh=mesh,
        scratch_types=[
            pltpu.VMEM((VOCAB,), jnp.float32),
            pltpu.VMEM((IDX_CHUNK,), jnp.int32),
            pltpu.VMEM((IDX_CHUNK,), jnp.int32),
            pltpu.VMEM((IDX_CHUNK,), jnp.float32),
            pltpu.VMEM((IDX_CHUNK,), jnp.float32),
            pltpu.SemaphoreType.DMA,
            pltpu.SemaphoreType.DMA,
            pltpu.SemaphoreType.DMA,
            pltpu.SemaphoreType.DMA,
            pltpu.SemaphoreType.DMA,
        ],
        compiler_params=pltpu.CompilerParams(use_tc_tiling_on_sc=True,
                                             needs_layout_passes=False),
    )
    def k(t2T_hbm, t1T_hbm, idx_hbm, o2T_hbm, o1T_hbm, row_v,
          idx_v0, idx_v1, out_v0, out_v1, sem_row, sem_i0, sem_i1,
          sem_o0, sem_o1):
        w = lax.axis_index("s") * 2 + lax.axis_index("c")
        idx_bufs, out_bufs = (idx_v0, idx_v1), (out_v0, out_v1)
        isems, osems = (sem_i0, sem_i1), (sem_o0, sem_o1)

        def gather_row(table_row_ref, f, out_row_ref):
            pltpu.sync_copy(table_row_ref, row_v)
            for ci in range(N_CHUNKS):
                b = ci % 2
                pltpu.sync_copy(
                    idx_hbm.at[f, pl.ds(ci * IDX_CHUNK, IDX_CHUNK)],
                    idx_bufs[b])

                @plsc.parallel_loop(0, IDX_CHUNK, 16, unroll=4)
                def _(j):
                    idx16 = idx_bufs[b][pl.ds(j, 16)]
                    out_bufs[b][pl.ds(j, 16)] = plsc.load_gather(
                        row_v, [idx16])

                pltpu.sync_copy(
                    out_bufs[b],
                    out_row_ref.at[pl.ds(ci * IDX_CHUNK, IDX_CHUNK)])

        @pl.loop(0, NUM_FIELDS)
        def _(f):
            r = f * EMB + w
            gather_row(t2T_hbm.at[r], f, o2T_hbm.at[r])

        @pl.when(w < NUM_FIELDS)
        def _():
            gather_row(t1T_hbm.at[w], w, o1T_hbm.at[w])

    return k(t2T, t1T, xT_idx)


BB = 2048


def _tc_body(xT_ref, fm1T_ref, w1_ref, b1_ref, g1_ref, be1_ref, rm1_ref,
             rv1_ref, w2_ref, b2_ref, g2_ref, be2_ref, rm2_ref, rv2_ref,
             w3_ref, b3_ref, out_ref):
    xT = xT_ref[...]

    sum_e = xT[0:EMB, :]
    sum_sq = sum_e * sum_e
    for f in range(1, NUM_FIELDS):
        v = xT[f * EMB:(f + 1) * EMB, :]
        sum_e = sum_e + v
        sum_sq = sum_sq + v * v
    fm2 = 0.5 * jnp.sum(sum_e * sum_e - sum_sq, axis=0, keepdims=True)

    fm1 = jnp.sum(fm1T_ref[...], axis=0, keepdims=True)

    dn = (((0,), (0,)), ((), ()))

    h = lax.dot_general(w1_ref[...], xT, dn,
                        preferred_element_type=jnp.float32) + b1_ref[...]
    h = (h - rm1_ref[...]) * (g1_ref[...] * lax.rsqrt(rv1_ref[...] + EPS)) + be1_ref[...]
    h = jnp.maximum(h, 0.0)
    h = lax.dot_general(w2_ref[...], h, dn,
                        preferred_element_type=jnp.float32) + b2_ref[...]
    h = (h - rm2_ref[...]) * (g2_ref[...] * lax.rsqrt(rv2_ref[...] + EPS)) + be2_ref[...]
    h = jnp.maximum(h, 0.0)
    dnn = lax.dot_general(w3_ref[...], h, dn,
                          preferred_element_type=jnp.float32) + b3_ref[...]

    out_ref[...] = jax.nn.sigmoid(fm1 + fm2 + dnn)


def _tc_head(xT, fm1T, W1, b1, g1, be1, rm1, rv1, W2, b2, g2, be2, rm2, rv2,
             W3, b3):
    grid = (BATCH // BB,)
    full = lambda shape: pl.BlockSpec(shape, lambda i: tuple(0 for _ in shape))
    return pl.pallas_call(
        _tc_body,
        grid=grid,
        in_specs=[
            pl.BlockSpec((D_IN, BB), lambda i: (0, i)),
            pl.BlockSpec((NUM_FIELDS, BB), lambda i: (0, i)),
            full((D_IN, H1)), full((H1, 1)), full((H1, 1)), full((H1, 1)),
            full((H1, 1)), full((H1, 1)),
            full((H1, H2)), full((H2, 1)), full((H2, 1)), full((H2, 1)),
            full((H2, 1)), full((H2, 1)),
            full((H2, 1)), full((1, 1)),
        ],
        out_specs=pl.BlockSpec((1, BB), lambda i: (0, i)),
        out_shape=jax.ShapeDtypeStruct((1, BATCH), jnp.float32),
    )(xT, fm1T, W1, b1, g1, be1, rm1, rv1, W2, b2, g2, be2, rm2, rv2, W3, b3)


def kernel(X_sparse, emb1, emb2, W1, b1, g1, be1, rm1, rv1, W2, b2, g2, be2,
           rm2, rv2, W3, b3):
    t2T = jnp.transpose(emb2, (0, 2, 1)).reshape(D_IN, VOCAB)
    t1T = jnp.transpose(emb1, (0, 2, 1)).reshape(NUM_FIELDS, VOCAB)
    xT_idx = jnp.transpose(X_sparse, (1, 0)).astype(jnp.int32)

    o2T, o1T = _sc_gather_t(t2T, t1T, xT_idx)

    r = lambda a: a.reshape(-1, 1)
    out_row = _tc_head(o2T, o1T, W1, r(b1), r(g1), r(be1), r(rm1), r(rv1),
                       W2, r(b2), r(g2), r(be2), r(rm2), r(rv2), r(W3), r(b3))
    return out_row.reshape(BATCH, 1)


# speedup vs baseline: 2.0015x; 1.2929x over previous
import jax
import jax.numpy as jnp
from jax import lax
from jax.experimental import pallas as pl
from jax.experimental.pallas import tpu as pltpu
from jax.experimental.pallas import tpu_sc as plsc

NUM_FIELDS = 26
VOCAB = 100000
EMB = 32
BATCH = 16384
D_IN = NUM_FIELDS * EMB
H1, H2 = 256, 128
EPS = 1e-5

IDX_CHUNK = 4096
N_CHUNKS = BATCH // IDX_CHUNK


def _sc_gather_t(t2T, t1T, xT_idx):
    mesh = plsc.VectorSubcoreMesh(core_axis_name="c", subcore_axis_name="s")

    @pl.kernel(
        out_type=(
            jax.ShapeDtypeStruct((D_IN, BATCH), jnp.float32),
            jax.ShapeDtypeStruct((NUM_FIELDS, BATCH), jnp.float32),
        ),
        mesh=mesh,
        scratch_types=[
            pltpu.VMEM((VOCAB,), jnp.float32),
            pltpu.VMEM((BATCH,), jnp.int32),
            pltpu.VMEM((IDX_CHUNK,), jnp.float32),
            pltpu.VMEM((IDX_CHUNK,), jnp.float32),
        ],
        compiler_params=pltpu.CompilerParams(use_tc_tiling_on_sc=True,
                                             needs_layout_passes=False),
    )
    def k(t2T_hbm, t1T_hbm, idx_hbm, o2T_hbm, o1T_hbm, row_v, idxrow_v,
          out_v0, out_v1):
        w = lax.axis_index("s") * 2 + lax.axis_index("c")
        out_bufs = (out_v0, out_v1)
        h = w // 16
        p = w % 16

        def gather_resident_idx(table_row_ref, out_row_ref):
            pltpu.sync_copy(table_row_ref, row_v)
            for ci in range(N_CHUNKS):
                b = ci % 2
                base = ci * IDX_CHUNK

                @plsc.parallel_loop(0, IDX_CHUNK, 16, unroll=4)
                def _(j):
                    idx16 = idxrow_v[pl.ds(base + j, 16)]
                    out_bufs[b][pl.ds(j, 16)] = plsc.load_gather(
                        row_v, [idx16])

                pltpu.sync_copy(out_bufs[b],
                                out_row_ref.at[pl.ds(base, IDX_CHUNK)])

        @pl.loop(0, NUM_FIELDS // 2)
        def _(j):
            f = h * (NUM_FIELDS // 2) + j
            pltpu.sync_copy(idx_hbm.at[f], idxrow_v)
            for d in range(2):
                r = f * EMB + 2 * p + d
                gather_resident_idx(t2T_hbm.at[r], o2T_hbm.at[r])

        @pl.when(p < NUM_FIELDS // 2)
        def _():
            f1 = h * (NUM_FIELDS // 2) + p
            pltpu.sync_copy(idx_hbm.at[f1], idxrow_v)
            gather_resident_idx(t1T_hbm.at[f1], o1T_hbm.at[f1])

    return k(t2T, t1T, xT_idx)


BB = 2048


def _tc_body(xT_ref, fm1T_ref, w1_ref, b1_ref, g1_ref, be1_ref, rm1_ref,
             rv1_ref, w2_ref, b2_ref, g2_ref, be2_ref, rm2_ref, rv2_ref,
             w3_ref, b3_ref, out_ref):
    xT = xT_ref[...]

    sum_e = xT[0:EMB, :]
    sum_sq = sum_e * sum_e
    for f in range(1, NUM_FIELDS):
        v = xT[f * EMB:(f + 1) * EMB, :]
        sum_e = sum_e + v
        sum_sq = sum_sq + v * v
    fm2 = 0.5 * jnp.sum(sum_e * sum_e - sum_sq, axis=0, keepdims=True)

    fm1 = jnp.sum(fm1T_ref[...], axis=0, keepdims=True)

    dn = (((0,), (0,)), ((), ()))

    h = lax.dot_general(w1_ref[...], xT, dn,
                        preferred_element_type=jnp.float32) + b1_ref[...]
    h = (h - rm1_ref[...]) * (g1_ref[...] * lax.rsqrt(rv1_ref[...] + EPS)) + be1_ref[...]
    h = jnp.maximum(h, 0.0)
    h = lax.dot_general(w2_ref[...], h, dn,
                        preferred_element_type=jnp.float32) + b2_ref[...]
    h = (h - rm2_ref[...]) * (g2_ref[...] * lax.rsqrt(rv2_ref[...] + EPS)) + be2_ref[...]
    h = jnp.maximum(h, 0.0)
    dnn = lax.dot_general(w3_ref[...], h, dn,
                          preferred_element_type=jnp.float32) + b3_ref[...]

    out_ref[...] = jax.nn.sigmoid(fm1 + fm2 + dnn)


def _tc_head(xT, fm1T, W1, b1, g1, be1, rm1, rv1, W2, b2, g2, be2, rm2, rv2,
             W3, b3):
    grid = (BATCH // BB,)
    full = lambda shape: pl.BlockSpec(shape, lambda i: tuple(0 for _ in shape))
    return pl.pallas_call(
        _tc_body,
        grid=grid,
        in_specs=[
            pl.BlockSpec((D_IN, BB), lambda i: (0, i)),
            pl.BlockSpec((NUM_FIELDS, BB), lambda i: (0, i)),
            full((D_IN, H1)), full((H1, 1)), full((H1, 1)), full((H1, 1)),
            full((H1, 1)), full((H1, 1)),
            full((H1, H2)), full((H2, 1)), full((H2, 1)), full((H2, 1)),
            full((H2, 1)), full((H2, 1)),
            full((H2, 1)), full((1, 1)),
        ],
        out_specs=pl.BlockSpec((1, BB), lambda i: (0, i)),
        out_shape=jax.ShapeDtypeStruct((1, BATCH), jnp.float32),
    )(xT, fm1T, W1, b1, g1, be1, rm1, rv1, W2, b2, g2, be2, rm2, rv2, W3, b3)


def kernel(X_sparse, emb1, emb2, W1, b1, g1, be1, rm1, rv1, W2, b2, g2, be2,
           rm2, rv2, W3, b3):
    t2T = jnp.transpose(emb2, (0, 2, 1)).reshape(D_IN, VOCAB)
    t1T = jnp.transpose(emb1, (0, 2, 1)).reshape(NUM_FIELDS, VOCAB)
    xT_idx = jnp.transpose(X_sparse, (1, 0)).astype(jnp.int32)

    o2T, o1T = _sc_gather_t(t2T, t1T, xT_idx)

    r = lambda a: a.reshape(-1, 1)
    out_row = _tc_head(o2T, o1T, W1, r(b1), r(g1), r(be1), r(rm1), r(rv1),
                       W2, r(b2), r(g2), r(be2), r(rm2), r(rv2), r(W3), r(b3))
    return out_row.reshape(BATCH, 1)


# speedup vs baseline: 2.1217x; 1.0600x over previous
import jax
import jax.numpy as jnp
from jax import lax
from jax.experimental import pallas as pl
from jax.experimental.pallas import tpu as pltpu
from jax.experimental.pallas import tpu_sc as plsc

NUM_FIELDS = 26
VOCAB = 100000
EMB = 32
BATCH = 16384
D_IN = NUM_FIELDS * EMB
H1, H2 = 256, 128
EPS = 1e-5

IDX_CHUNK = 4096
N_CHUNKS = BATCH // IDX_CHUNK


def _sc_gather_t(t2T, t1T, xT_idx):
    mesh = plsc.VectorSubcoreMesh(core_axis_name="c", subcore_axis_name="s")

    @pl.kernel(
        out_type=(
            jax.ShapeDtypeStruct((D_IN, BATCH), jnp.float32),
            jax.ShapeDtypeStruct((NUM_FIELDS, BATCH), jnp.float32),
        ),
        mesh=mesh,
        scratch_types=[
            pltpu.VMEM((VOCAB,), jnp.float32),
            pltpu.VMEM((BATCH,), jnp.int32),
            pltpu.VMEM((IDX_CHUNK,), jnp.float32),
            pltpu.VMEM((IDX_CHUNK,), jnp.float32),
            pltpu.SemaphoreType.DMA,
            pltpu.SemaphoreType.DMA,
        ],
        compiler_params=pltpu.CompilerParams(use_tc_tiling_on_sc=True,
                                             needs_layout_passes=False),
    )
    def k(t2T_hbm, t1T_hbm, idx_hbm, o2T_hbm, o1T_hbm, row_v, idxrow_v,
          out_v0, out_v1, sem_o0, sem_o1):
        w = lax.axis_index("s") * 2 + lax.axis_index("c")
        out_bufs = (out_v0, out_v1)
        osems = (sem_o0, sem_o1)
        h = w // 16
        p = w % 16

        def gather_resident_idx(table_row_ref, out_row_ref):
            pltpu.sync_copy(table_row_ref, row_v)
            descs = []
            for ci in range(N_CHUNKS):
                b = ci % 2
                base = ci * IDX_CHUNK

                @plsc.parallel_loop(0, IDX_CHUNK, 16, unroll=4)
                def _(j):
                    idx16 = idxrow_v[pl.ds(base + j, 16)]
                    out_bufs[b][pl.ds(j, 16)] = plsc.load_gather(
                        row_v, [idx16])

                descs.append(pltpu.async_copy(
                    out_bufs[b], out_row_ref.at[pl.ds(base, IDX_CHUNK)],
                    osems[b]))
                if ci >= 1:
                    descs[ci - 1].wait()
            descs[N_CHUNKS - 1].wait()

        @pl.loop(0, NUM_FIELDS // 2)
        def _(j):
            f = h * (NUM_FIELDS // 2) + j
            pltpu.sync_copy(idx_hbm.at[f], idxrow_v)
            for d in range(2):
                r = f * EMB + 2 * p + d
                gather_resident_idx(t2T_hbm.at[r], o2T_hbm.at[r])

        @pl.when(p < NUM_FIELDS // 2)
        def _():
            f1 = h * (NUM_FIELDS // 2) + p
            pltpu.sync_copy(idx_hbm.at[f1], idxrow_v)
            gather_resident_idx(t1T_hbm.at[f1], o1T_hbm.at[f1])

    return k(t2T, t1T, xT_idx)


BB = 2048


def _tc_body(xT_ref, fm1T_ref, w1_ref, b1_ref, g1_ref, be1_ref, rm1_ref,
             rv1_ref, w2_ref, b2_ref, g2_ref, be2_ref, rm2_ref, rv2_ref,
             w3_ref, b3_ref, out_ref):
    xT = xT_ref[...]

    sum_e = xT[0:EMB, :]
    sum_sq = sum_e * sum_e
    for f in range(1, NUM_FIELDS):
        v = xT[f * EMB:(f + 1) * EMB, :]
        sum_e = sum_e + v
        sum_sq = sum_sq + v * v
    fm2 = 0.5 * jnp.sum(sum_e * sum_e - sum_sq, axis=0, keepdims=True)

    fm1 = jnp.sum(fm1T_ref[...], axis=0, keepdims=True)

    dn = (((0,), (0,)), ((), ()))

    h = lax.dot_general(w1_ref[...], xT, dn,
                        preferred_element_type=jnp.float32) + b1_ref[...]
    h = (h - rm1_ref[...]) * (g1_ref[...] * lax.rsqrt(rv1_ref[...] + EPS)) + be1_ref[...]
    h = jnp.maximum(h, 0.0)
    h = lax.dot_general(w2_ref[...], h, dn,
                        preferred_element_type=jnp.float32) + b2_ref[...]
    h = (h - rm2_ref[...]) * (g2_ref[...] * lax.rsqrt(rv2_ref[...] + EPS)) + be2_ref[...]
    h = jnp.maximum(h, 0.0)
    dnn = lax.dot_general(w3_ref[...], h, dn,
                          preferred_element_type=jnp.float32) + b3_ref[...]

    out_ref[...] = jax.nn.sigmoid(fm1 + fm2 + dnn)


def _tc_head(xT, fm1T, W1, b1, g1, be1, rm1, rv1, W2, b2, g2, be2, rm2, rv2,
             W3, b3):
    grid = (BATCH // BB,)
    full = lambda shape: pl.BlockSpec(shape, lambda i: tuple(0 for _ in shape))
    return pl.pallas_call(
        _tc_body,
        grid=grid,
        in_specs=[
            pl.BlockSpec((D_IN, BB), lambda i: (0, i)),
            pl.BlockSpec((NUM_FIELDS, BB), lambda i: (0, i)),
            full((D_IN, H1)), full((H1, 1)), full((H1, 1)), full((H1, 1)),
            full((H1, 1)), full((H1, 1)),
            full((H1, H2)), full((H2, 1)), full((H2, 1)), full((H2, 1)),
            full((H2, 1)), full((H2, 1)),
            full((H2, 1)), full((1, 1)),
        ],
        out_specs=pl.BlockSpec((1, BB), lambda i: (0, i)),
        out_shape=jax.ShapeDtypeStruct((1, BATCH), jnp.float32),
    )(xT, fm1T, W1, b1, g1, be1, rm1, rv1, W2, b2, g2, be2, rm2, rv2, W3, b3)


def kernel(X_sparse, emb1, emb2, W1, b1, g1, be1, rm1, rv1, W2, b2, g2, be2,
           rm2, rv2, W3, b3):
    t2T = jnp.transpose(emb2, (0, 2, 1)).reshape(D_IN, VOCAB)
    t1T = jnp.transpose(emb1, (0, 2, 1)).reshape(NUM_FIELDS, VOCAB)
    xT_idx = jnp.transpose(X_sparse, (1, 0)).astype(jnp.int32)

    o2T, o1T = _sc_gather_t(t2T, t1T, xT_idx)

    r = lambda a: a.reshape(-1, 1)
    out_row = _tc_head(o2T, o1T, W1, r(b1), r(g1), r(be1), r(rm1), r(rv1),
                       W2, r(b2), r(g2), r(be2), r(rm2), r(rv2), r(W3), r(b3))
    return out_row.reshape(BATCH, 1)


# speedup vs baseline: 2.1319x; 1.0048x over previous
import jax
import jax.numpy as jnp
from jax import lax
from jax.experimental import pallas as pl
from jax.experimental.pallas import tpu as pltpu
from jax.experimental.pallas import tpu_sc as plsc

NUM_FIELDS = 26
VOCAB = 100000
EMB = 32
BATCH = 16384
D_IN = NUM_FIELDS * EMB
H1, H2 = 256, 128
EPS = 1e-5

IDX_CHUNK = 4096
N_CHUNKS = BATCH // IDX_CHUNK


def _sc_gather_t(t2T, t1T, xT_idx):
    mesh = plsc.VectorSubcoreMesh(core_axis_name="c", subcore_axis_name="s")

    @pl.kernel(
        out_type=(
            jax.ShapeDtypeStruct((D_IN, BATCH), jnp.float32),
            jax.ShapeDtypeStruct((NUM_FIELDS, BATCH), jnp.float32),
        ),
        mesh=mesh,
        scratch_types=[
            pltpu.VMEM((VOCAB,), jnp.float32),
            pltpu.VMEM((BATCH,), jnp.int32),
            pltpu.VMEM((IDX_CHUNK,), jnp.float32),
            pltpu.VMEM((IDX_CHUNK,), jnp.float32),
            pltpu.SemaphoreType.DMA,
            pltpu.SemaphoreType.DMA,
        ],
        compiler_params=pltpu.CompilerParams(use_tc_tiling_on_sc=True,
                                             needs_layout_passes=False),
    )
    def k(t2T_hbm, t1T_hbm, idx_hbm, o2T_hbm, o1T_hbm, row_v, idxrow_v,
          out_v0, out_v1, sem_o0, sem_o1):
        w = lax.axis_index("s") * 2 + lax.axis_index("c")
        out_bufs = (out_v0, out_v1)
        osems = (sem_o0, sem_o1)
        h = w // 16
        p = w % 16

        def gather_resident_idx(table_row_ref, out_row_ref):
            pltpu.sync_copy(table_row_ref, row_v)
            descs = []
            for ci in range(N_CHUNKS):
                b = ci % 2
                base = ci * IDX_CHUNK

                @plsc.parallel_loop(0, IDX_CHUNK, 16, unroll=4)
                def _(j):
                    idx16 = idxrow_v[pl.ds(base + j, 16)]
                    out_bufs[b][pl.ds(j, 16)] = plsc.load_gather(
                        row_v, [idx16])

                descs.append(pltpu.async_copy(
                    out_bufs[b], out_row_ref.at[pl.ds(base, IDX_CHUNK)],
                    osems[b]))
                if ci >= 1:
                    descs[ci - 1].wait()
            descs[N_CHUNKS - 1].wait()

        @pl.loop(0, NUM_FIELDS // 2)
        def _(j):
            f = h * (NUM_FIELDS // 2) + j
            pltpu.sync_copy(idx_hbm.at[f], idxrow_v)
            for d in range(2):
                r = f * EMB + 2 * p + d
                gather_resident_idx(t2T_hbm.at[r], o2T_hbm.at[r])

        @pl.when(p < NUM_FIELDS // 2)
        def _():
            f1 = h * (NUM_FIELDS // 2) + p
            pltpu.sync_copy(idx_hbm.at[f1], idxrow_v)
            gather_resident_idx(t1T_hbm.at[f1], o1T_hbm.at[f1])

    return k(t2T, t1T, xT_idx)


BB = 2048


def _tc_body(xT_ref, fm1T_ref, w1_ref, b1_ref, g1_ref, be1_ref, rm1_ref,
             rv1_ref, w2_ref, b2_ref, g2_ref, be2_ref, rm2_ref, rv2_ref,
             w3_ref, b3_ref, out_ref):
    xT = xT_ref[...]

    sum_e = xT[0:EMB, :]
    sum_sq = sum_e * sum_e
    for f in range(1, NUM_FIELDS):
        v = xT[f * EMB:(f + 1) * EMB, :]
        sum_e = sum_e + v
        sum_sq = sum_sq + v * v
    fm2 = 0.5 * jnp.sum(sum_e * sum_e - sum_sq, axis=0, keepdims=True)

    fm1 = jnp.sum(fm1T_ref[...], axis=0, keepdims=True)

    dn = (((0,), (0,)), ((), ()))

    bf = jnp.bfloat16
    h = lax.dot_general(w1_ref[...], xT.astype(bf), dn,
                        preferred_element_type=jnp.float32) + b1_ref[...]
    h = (h - rm1_ref[...]) * (g1_ref[...] * lax.rsqrt(rv1_ref[...] + EPS)) + be1_ref[...]
    h = jnp.maximum(h, 0.0)
    h = lax.dot_general(w2_ref[...], h.astype(bf), dn,
                        preferred_element_type=jnp.float32) + b2_ref[...]
    h = (h - rm2_ref[...]) * (g2_ref[...] * lax.rsqrt(rv2_ref[...] + EPS)) + be2_ref[...]
    h = jnp.maximum(h, 0.0)
    dnn = lax.dot_general(w3_ref[...], h.astype(bf), dn,
                          preferred_element_type=jnp.float32) + b3_ref[...]

    out_ref[...] = jax.nn.sigmoid(fm1 + fm2 + dnn)


def _tc_head(xT, fm1T, W1, b1, g1, be1, rm1, rv1, W2, b2, g2, be2, rm2, rv2,
             W3, b3):
    grid = (BATCH // BB,)
    full = lambda shape: pl.BlockSpec(shape, lambda i: tuple(0 for _ in shape))
    return pl.pallas_call(
        _tc_body,
        grid=grid,
        in_specs=[
            pl.BlockSpec((D_IN, BB), lambda i: (0, i)),
            pl.BlockSpec((NUM_FIELDS, BB), lambda i: (0, i)),
            full((D_IN, H1)), full((H1, 1)), full((H1, 1)), full((H1, 1)),
            full((H1, 1)), full((H1, 1)),
            full((H1, H2)), full((H2, 1)), full((H2, 1)), full((H2, 1)),
            full((H2, 1)), full((H2, 1)),
            full((H2, 1)), full((1, 1)),
        ],
        out_specs=pl.BlockSpec((1, BB), lambda i: (0, i)),
        out_shape=jax.ShapeDtypeStruct((1, BATCH), jnp.float32),
    )(xT, fm1T, W1, b1, g1, be1, rm1, rv1, W2, b2, g2, be2, rm2, rv2, W3, b3)


def kernel(X_sparse, emb1, emb2, W1, b1, g1, be1, rm1, rv1, W2, b2, g2, be2,
           rm2, rv2, W3, b3):
    t2T = jnp.transpose(emb2, (0, 2, 1)).reshape(D_IN, VOCAB)
    t1T = jnp.transpose(emb1, (0, 2, 1)).reshape(NUM_FIELDS, VOCAB)
    xT_idx = jnp.transpose(X_sparse, (1, 0)).astype(jnp.int32)

    o2T, o1T = _sc_gather_t(t2T, t1T, xT_idx)

    r = lambda a: a.reshape(-1, 1)
    bf = lambda a: a.astype(jnp.bfloat16)
    out_row = _tc_head(o2T, o1T, bf(W1), r(b1), r(g1), r(be1), r(rm1), r(rv1),
                       bf(W2), r(b2), r(g2), r(be2), r(rm2), r(rv2),
                       bf(r(W3)), r(b3))
    return out_row.reshape(BATCH, 1)


# speedup vs baseline: 2.2448x; 1.0530x over previous
import jax
import jax.numpy as jnp
from jax import lax
from jax.experimental import pallas as pl
from jax.experimental.pallas import tpu as pltpu
from jax.experimental.pallas import tpu_sc as plsc

NUM_FIELDS = 26
VOCAB = 100000
EMB = 32
BATCH = 16384
D_IN = NUM_FIELDS * EMB
H1, H2 = 256, 128
EPS = 1e-5

IDX_CHUNK = 4096
N_CHUNKS = BATCH // IDX_CHUNK


def _sc_gather_t(t2T, t1T, xT_idx):
    mesh = plsc.VectorSubcoreMesh(core_axis_name="c", subcore_axis_name="s")

    @pl.kernel(
        out_type=(
            jax.ShapeDtypeStruct((D_IN, BATCH), jnp.float32),
            jax.ShapeDtypeStruct((NUM_FIELDS, BATCH), jnp.float32),
        ),
        mesh=mesh,
        scratch_types=[
            pltpu.VMEM((VOCAB,), jnp.float32),
            pltpu.VMEM((BATCH,), jnp.int32),
            pltpu.VMEM((IDX_CHUNK,), jnp.float32),
            pltpu.VMEM((IDX_CHUNK,), jnp.float32),
            pltpu.SemaphoreType.DMA,
            pltpu.SemaphoreType.DMA,
        ],
        compiler_params=pltpu.CompilerParams(use_tc_tiling_on_sc=True,
                                             needs_layout_passes=False),
    )
    def k(t2T_hbm, t1T_hbm, idx_hbm, o2T_hbm, o1T_hbm, row_v, idxrow_v,
          out_v0, out_v1, sem_o0, sem_o1):
        w = lax.axis_index("s") * 2 + lax.axis_index("c")
        out_bufs = (out_v0, out_v1)
        osems = (sem_o0, sem_o1)
        h = w // 16
        p = w % 16

        def gather_resident_idx(table_row_ref, out_row_ref):
            pltpu.sync_copy(table_row_ref, row_v)
            descs = []
            for ci in range(N_CHUNKS):
                b = ci % 2
                base = ci * IDX_CHUNK

                @plsc.parallel_loop(0, IDX_CHUNK, 16, unroll=4)
                def _(j):
                    idx16 = idxrow_v[pl.ds(base + j, 16)]
                    out_bufs[b][pl.ds(j, 16)] = plsc.load_gather(
                        row_v, [idx16])

                descs.append(pltpu.async_copy(
                    out_bufs[b], out_row_ref.at[pl.ds(base, IDX_CHUNK)],
                    osems[b]))
                if ci >= 1:
                    descs[ci - 1].wait()
            descs[N_CHUNKS - 1].wait()

        @pl.loop(0, NUM_FIELDS // 2)
        def _(j):
            f = h * (NUM_FIELDS // 2) + j
            pltpu.sync_copy(idx_hbm.at[f], idxrow_v)
            for d in range(2):
                r = f * EMB + 2 * p + d
                gather_resident_idx(t2T_hbm.at[r], o2T_hbm.at[r])

        @pl.when(p < NUM_FIELDS // 2)
        def _():
            f1 = h * (NUM_FIELDS // 2) + p
            pltpu.sync_copy(idx_hbm.at[f1], idxrow_v)
            gather_resident_idx(t1T_hbm.at[f1, 0], o1T_hbm.at[f1])

    return k(t2T, t1T, xT_idx)


BB = 4096


def _tc_body(xT_ref, fm1T_ref, w1_ref, b1_ref, g1_ref, be1_ref, rm1_ref,
             rv1_ref, w2_ref, b2_ref, g2_ref, be2_ref, rm2_ref, rv2_ref,
             w3_ref, b3_ref, out_ref):
    xT = xT_ref[...]

    sum_e = xT[0:EMB, :]
    sum_sq = sum_e * sum_e
    for f in range(1, NUM_FIELDS):
        v = xT[f * EMB:(f + 1) * EMB, :]
        sum_e = sum_e + v
        sum_sq = sum_sq + v * v
    fm2 = 0.5 * jnp.sum(sum_e * sum_e - sum_sq, axis=0, keepdims=True)

    fm1 = jnp.sum(fm1T_ref[...], axis=0, keepdims=True)

    dn = (((0,), (0,)), ((), ()))

    h = lax.dot_general(w1_ref[...], xT, dn,
                        preferred_element_type=jnp.float32) + b1_ref[...]
    h = (h - rm1_ref[...]) * (g1_ref[...] * lax.rsqrt(rv1_ref[...] + EPS)) + be1_ref[...]
    h = jnp.maximum(h, 0.0)
    h = lax.dot_general(w2_ref[...], h, dn,
                        preferred_element_type=jnp.float32) + b2_ref[...]
    h = (h - rm2_ref[...]) * (g2_ref[...] * lax.rsqrt(rv2_ref[...] + EPS)) + be2_ref[...]
    h = jnp.maximum(h, 0.0)
    dnn = lax.dot_general(w3_ref[...], h, dn,
                          preferred_element_type=jnp.float32) + b3_ref[...]

    out_ref[...] = jax.nn.sigmoid(fm1 + fm2 + dnn)


def _tc_head(xT, fm1T, W1, b1, g1, be1, rm1, rv1, W2, b2, g2, be2, rm2, rv2,
             W3, b3):
    grid = (BATCH // BB,)
    full = lambda shape: pl.BlockSpec(shape, lambda i: tuple(0 for _ in shape))
    return pl.pallas_call(
        _tc_body,
        grid=grid,
        in_specs=[
            pl.BlockSpec((D_IN, BB), lambda i: (0, i)),
            pl.BlockSpec((NUM_FIELDS, BB), lambda i: (0, i)),
            full((D_IN, H1)), full((H1, 1)), full((H1, 1)), full((H1, 1)),
            full((H1, 1)), full((H1, 1)),
            full((H1, H2)), full((H2, 1)), full((H2, 1)), full((H2, 1)),
            full((H2, 1)), full((H2, 1)),
            full((H2, 1)), full((1, 1)),
        ],
        out_specs=pl.BlockSpec((1, BB), lambda i: (0, i)),
        out_shape=jax.ShapeDtypeStruct((1, BATCH), jnp.float32),
    )(xT, fm1T, W1, b1, g1, be1, rm1, rv1, W2, b2, g2, be2, rm2, rv2, W3, b3)


def kernel(X_sparse, emb1, emb2, W1, b1, g1, be1, rm1, rv1, W2, b2, g2, be2,
           rm2, rv2, W3, b3):
    t2T = jnp.transpose(emb2, (0, 2, 1)).reshape(D_IN, VOCAB)
    t1T = jnp.transpose(emb1, (0, 2, 1))
    xT_idx = jnp.transpose(X_sparse, (1, 0)).astype(jnp.int32)

    o2T, o1T = _sc_gather_t(t2T, t1T, xT_idx)

    r = lambda a: a.reshape(-1, 1)
    out_row = _tc_head(o2T, o1T, W1, r(b1), r(g1), r(be1), r(rm1), r(rv1),
                       W2, r(b2), r(g2), r(be2), r(rm2), r(rv2), r(W3), r(b3))
    return out_row.reshape(BATCH, 1)
